# Initial kernel scaffold; baseline (speedup 1.0000x reference)
import jax
import jax.numpy as jnp
from jax.experimental import pallas as pl


def kernel(tokens, prompt_table, W1, b1, W2, b2):
    raise NotImplementedError("write your pallas kernel here")


# speedup vs baseline: 1.2121x; 1.2121x over previous
import functools

import jax
import jax.numpy as jnp
from jax import lax
from jax.experimental import pallas as pl
from jax.experimental.pallas import tpu as pltpu
from jax.experimental.pallas import tpu_sc as plsc

_V = 32000
_B = 16
_P = 50
_R = _B * _P
_NPROMPT = 400
_H = 1024
_PH = 512
_L2 = 48
_NH = 16
_DH = 64
_NW = 32
_RPAD = 1024
_RPW = _RPAD // _NW


def _mlp1(pt, w1, b1r):
    def body(pt_ref, w1_ref, b1_ref, a_ref):
        p = pt_ref[...].astype(jnp.bfloat16)
        w = w1_ref[...].astype(jnp.bfloat16)
        acc = lax.dot_general(p, w, (((1,), (0,)), ((), ())),
                              preferred_element_type=jnp.float32)
        a_ref[...] = jnp.tanh(acc + b1_ref[...])

    return pl.pallas_call(
        body,
        out_shape=jax.ShapeDtypeStruct((_NPROMPT, _PH), jnp.float32),
    )(pt, w1, b1r)


def _sc_gather(a, idx):
    mesh = plsc.VectorSubcoreMesh(core_axis_name="c", subcore_axis_name="s")

    @functools.partial(
        pl.kernel,
        mesh=mesh,
        out_type=jax.ShapeDtypeStruct((_RPAD, _PH), jnp.float32),
        scratch_types=[
            pltpu.VMEM((_RPW,), jnp.int32),
            pltpu.VMEM((_RPW, _PH), jnp.float32),
            pltpu.SemaphoreType.DMA,
        ],
    )
    def k(a_hbm, idx_hbm, out_hbm, idx_v, rows_v, sem):
        wid = lax.axis_index("s") * 2 + lax.axis_index("c")
        base = wid * _RPW
        pltpu.sync_copy(idx_hbm.at[pl.ds(base, _RPW)], idx_v)
        pltpu.async_copy(a_hbm.at[idx_v], rows_v, sem).wait()
        pltpu.sync_copy(rows_v, out_hbm.at[pl.ds(base, _RPW)])

    return k(a, idx)


def _mlp2(g, w2, b2r):
    def body(g_ref, w2_ref, b2_ref, out_ref):
        w = w2_ref[...].astype(jnp.bfloat16)
        bias = b2_ref[0]
        for b in range(_B):
            gb = g_ref[b]
            m = lax.dot_general(gb, w, (((1,), (0,)), ((), ())),
                                preferred_element_type=jnp.float32) + bias
            for j in range(8):
                out_ref[0, b, j] = m[:, _DH * j:_DH * (j + 1)]

    return pl.pallas_call(
        body,
        grid=(_L2, 2),
        in_specs=[
            pl.BlockSpec((_B, _P, _PH), lambda i, j: (0, 0, 0)),
            pl.BlockSpec((_PH, 512), lambda i, j: (0, i * 2 + j)),
            pl.BlockSpec((1, 1, 512), lambda i, j: (i * 2 + j, 0, 0)),
        ],
        out_specs=pl.BlockSpec((1, _B, 8, _P, _DH),
                               lambda i, j: (i, 0, j, 0, 0)),
        out_shape=jax.ShapeDtypeStruct((_L2, _B, _NH, _P, _DH), jnp.float32),
        compiler_params=pltpu.CompilerParams(
            dimension_semantics=("arbitrary", "arbitrary")),
    )(g, w2, b2r)


def kernel(tokens, prompt_table, W1, b1, W2, b2):
    idx = jnp.zeros((_RPAD,), jnp.int32).at[:_R].set(tokens.reshape(_R) - _V)
    a = _mlp1(prompt_table, W1, b1.reshape(1, _PH))
    g = _sc_gather(a, idx)
    gb = g[:_R].astype(jnp.bfloat16).reshape(_B, _P, _PH)
    return _mlp2(gb, W2, b2.reshape(96, 1, 512))


# speedup vs baseline: 2.2956x; 1.8939x over previous
import functools

import jax
import jax.numpy as jnp
from jax import lax
from jax.experimental import pallas as pl
from jax.experimental.pallas import tpu as pltpu
from jax.experimental.pallas import tpu_sc as plsc

_V = 32000
_B = 16
_P = 50
_R = _B * _P
_NPROMPT = 400
_H = 1024
_PH = 512
_L2 = 48
_NH = 16
_DH = 64
_NW = 32
_RPAD = 1024
_RPW = _RPAD // _NW


def _mlp1(pt, w1, b1r):
    def body(pt_ref, w1_ref, b1_ref, a_ref):
        p = pt_ref[...].astype(jnp.bfloat16)
        w = w1_ref[...].astype(jnp.bfloat16)
        acc = lax.dot_general(p, w, (((1,), (0,)), ((), ())),
                              preferred_element_type=jnp.float32)
        a_ref[...] = jnp.tanh(acc + b1_ref[...])

    return pl.pallas_call(
        body,
        out_shape=jax.ShapeDtypeStruct((_NPROMPT, _PH), jnp.float32),
    )(pt, w1, b1r)


def _sc_gather(a, idx):
    mesh = plsc.VectorSubcoreMesh(core_axis_name="c", subcore_axis_name="s")

    @functools.partial(
        pl.kernel,
        mesh=mesh,
        out_type=jax.ShapeDtypeStruct((_RPAD, _PH), jnp.float32),
        scratch_types=[
            pltpu.VMEM((_RPW,), jnp.int32),
            pltpu.VMEM((_RPW, _PH), jnp.float32),
            pltpu.SemaphoreType.DMA,
        ],
    )
    def k(a_hbm, idx_hbm, out_hbm, idx_v, rows_v, sem):
        wid = lax.axis_index("s") * 2 + lax.axis_index("c")
        base = wid * _RPW
        pltpu.sync_copy(idx_hbm.at[pl.ds(base, _RPW)], idx_v)
        pltpu.async_copy(a_hbm.at[idx_v], rows_v, sem).wait()
        pltpu.sync_copy(rows_v, out_hbm.at[pl.ds(base, _RPW)])

    return k(a, idx)


def _mlp2(g, w2, b2r):
    def body(g_ref, w2_ref, b2_ref, out_ref):
        w = w2_ref[...].astype(jnp.bfloat16)
        bias = b2_ref[0]
        for b in range(_B):
            gb = g_ref[b]
            m = lax.dot_general(gb, w, (((1,), (0,)), ((), ())),
                                preferred_element_type=jnp.float32) + bias
            out_ref[0, pl.ds(b * _P, _P)] = m.reshape(_P, 8, _DH)

    return pl.pallas_call(
        body,
        grid=(_L2, 2),
        in_specs=[
            pl.BlockSpec((_B, _P, _PH), lambda i, j: (0, 0, 0)),
            pl.BlockSpec((_PH, 512), lambda i, j: (0, i * 2 + j)),
            pl.BlockSpec((1, 1, 512), lambda i, j: (i * 2 + j, 0, 0)),
        ],
        out_specs=pl.BlockSpec((1, _R, 8, _DH),
                               lambda i, j: (i, 0, j, 0)),
        out_shape=jax.ShapeDtypeStruct((_L2, _R, _NH, _DH), jnp.float32),
        compiler_params=pltpu.CompilerParams(
            dimension_semantics=("arbitrary", "arbitrary")),
    )(g, w2, b2r)


def kernel(tokens, prompt_table, W1, b1, W2, b2):
    idx = jnp.zeros((_RPAD,), jnp.int32).at[:_R].set(tokens.reshape(_R) - _V)
    a = _mlp1(prompt_table, W1, b1.reshape(1, _PH))
    g = _sc_gather(a, idx)
    gb = g[:_R].astype(jnp.bfloat16).reshape(_B, _P, _PH)
    out4 = _mlp2(gb, W2, b2.reshape(96, 1, 512))
    return out4.reshape(_L2, _B, _P, _NH, _DH).transpose(0, 1, 3, 2, 4)


# speedup vs baseline: 2.6772x; 1.1662x over previous
import functools

import jax
import jax.numpy as jnp
from jax import lax
from jax.experimental import pallas as pl
from jax.experimental.pallas import tpu as pltpu
from jax.experimental.pallas import tpu_sc as plsc

_V = 32000
_B = 16
_P = 50
_R = _B * _P
_NPROMPT = 400
_H = 1024
_PH = 512
_L2 = 48
_NH = 16
_DH = 64
_NW = 32
_RPAD = 1024
_RPW = _RPAD // _NW


def _mlp1(pt, w1, b1r):
    def body(pt_ref, w1_ref, b1_ref, a_ref):
        p = pt_ref[...].astype(jnp.bfloat16)
        w = w1_ref[...].astype(jnp.bfloat16)
        acc = lax.dot_general(p, w, (((1,), (0,)), ((), ())),
                              preferred_element_type=jnp.float32)
        a_ref[...] = jnp.tanh(acc + b1_ref[...])

    return pl.pallas_call(
        body,
        out_shape=jax.ShapeDtypeStruct((_NPROMPT, _PH), jnp.float32),
    )(pt, w1, b1r)


def _sc_gather(a, idx):
    mesh = plsc.VectorSubcoreMesh(core_axis_name="c", subcore_axis_name="s")

    @functools.partial(
        pl.kernel,
        mesh=mesh,
        out_type=jax.ShapeDtypeStruct((_RPAD, _PH), jnp.float32),
        scratch_types=[
            pltpu.VMEM((_RPW,), jnp.int32),
            pltpu.VMEM((_RPW, _PH), jnp.float32),
            pltpu.SemaphoreType.DMA,
        ],
    )
    def k(a_hbm, idx_hbm, out_hbm, idx_v, rows_v, sem):
        wid = lax.axis_index("s") * 2 + lax.axis_index("c")
        base = wid * _RPW
        pltpu.sync_copy(idx_hbm.at[pl.ds(base, _RPW)], idx_v)
        pltpu.async_copy(a_hbm.at[idx_v], rows_v, sem).wait()
        pltpu.sync_copy(rows_v, out_hbm.at[pl.ds(base, _RPW)])

    return k(a, idx)


def _mlp2(g, w2, b2r):
    def body(g_ref, w2_ref, b2_ref, out_ref):
        w = w2_ref[...].astype(jnp.bfloat16)
        bias = b2_ref[0]
        for b in range(_B):
            gb = g_ref[b]
            m = lax.dot_general(gb, w, (((1,), (0,)), ((), ())),
                                preferred_element_type=jnp.float32) + bias
            out_ref[0, pl.ds(b * _P, _P)] = m.reshape(_P, _NH, _DH)

    return pl.pallas_call(
        body,
        grid=(_L2,),
        in_specs=[
            pl.BlockSpec((_B, _P, _PH), lambda i: (0, 0, 0)),
            pl.BlockSpec((_PH, _NH * _DH), lambda i: (0, i)),
            pl.BlockSpec((1, 1, _NH * _DH), lambda i: (i, 0, 0)),
        ],
        out_specs=pl.BlockSpec((1, _R, _NH, _DH),
                               lambda i: (i, 0, 0, 0)),
        out_shape=jax.ShapeDtypeStruct((_L2, _R, _NH, _DH), jnp.float32),
        compiler_params=pltpu.CompilerParams(
            dimension_semantics=("arbitrary",)),
    )(g, w2, b2r)


def kernel(tokens, prompt_table, W1, b1, W2, b2):
    idx = jnp.zeros((_RPAD,), jnp.int32).at[:_R].set(tokens.reshape(_R) - _V)
    a = _mlp1(prompt_table, W1, b1.reshape(1, _PH))
    g = _sc_gather(a, idx)
    gb = g[:_R].astype(jnp.bfloat16).reshape(_B, _P, _PH)
    out4 = _mlp2(gb, W2, b2.reshape(_L2, 1, _NH * _DH))
    return out4.reshape(_L2, _B, _P, _NH, _DH).transpose(0, 1, 3, 2, 4)


# speedup vs baseline: 2.7039x; 1.0100x over previous
import functools

import jax
import jax.numpy as jnp
from jax import lax
from jax.experimental import pallas as pl
from jax.experimental.pallas import tpu as pltpu
from jax.experimental.pallas import tpu_sc as plsc

_V = 32000
_B = 16
_P = 50
_R = _B * _P
_NPROMPT = 400
_H = 1024
_PH = 512
_L2 = 48
_NH = 16
_DH = 64
_NW = 32
_RPB = 64
_RPAD = _B * _RPB
_RPW = _RPAD // _NW


def _mlp1(pt, w1, b1r):
    def body(pt_ref, w1_ref, b1_ref, a_ref):
        p = pt_ref[...].astype(jnp.bfloat16)
        w = w1_ref[...].astype(jnp.bfloat16)
        acc = lax.dot_general(p, w, (((1,), (0,)), ((), ())),
                              preferred_element_type=jnp.float32)
        a_ref[...] = jnp.tanh(acc + b1_ref[...])

    return pl.pallas_call(
        body,
        out_shape=jax.ShapeDtypeStruct((_NPROMPT, _PH), jnp.float32),
    )(pt, w1, b1r)


def _sc_gather(a, idx):
    mesh = plsc.VectorSubcoreMesh(core_axis_name="c", subcore_axis_name="s")

    @functools.partial(
        pl.kernel,
        mesh=mesh,
        out_type=jax.ShapeDtypeStruct((_RPAD, _PH), jnp.float32),
        scratch_types=[
            pltpu.VMEM((_RPW,), jnp.int32),
            pltpu.VMEM((_RPW, _PH), jnp.float32),
            pltpu.SemaphoreType.DMA,
        ],
    )
    def k(a_hbm, idx_hbm, out_hbm, idx_v, rows_v, sem):
        wid = lax.axis_index("s") * 2 + lax.axis_index("c")
        base = wid * _RPW
        pltpu.sync_copy(idx_hbm.at[pl.ds(base, _RPW)], idx_v)
        pltpu.async_copy(a_hbm.at[idx_v], rows_v, sem).wait()
        pltpu.sync_copy(rows_v, out_hbm.at[pl.ds(base, _RPW)])

    return k(a, idx)


def _mlp2(g, w2, b2r):
    def body(g_ref, w2_ref, b2_ref, out_ref, gbf):
        i = pl.program_id(0)

        @pl.when(i == 0)
        def _():
            gbf[...] = g_ref[...].astype(jnp.bfloat16)

        w = w2_ref[...].astype(jnp.bfloat16)
        bias = b2_ref[0]
        for b in range(_B):
            gb = gbf[b]
            m = lax.dot_general(gb, w, (((1,), (0,)), ((), ())),
                                preferred_element_type=jnp.float32) + bias
            out_ref[0, pl.ds(b * _P, _P)] = m[:_P].reshape(_P, _NH, _DH)

    return pl.pallas_call(
        body,
        grid=(_L2,),
        in_specs=[
            pl.BlockSpec((_B, _RPB, _PH), lambda i: (0, 0, 0)),
            pl.BlockSpec((_PH, _NH * _DH), lambda i: (0, i)),
            pl.BlockSpec((1, 1, _NH * _DH), lambda i: (i, 0, 0)),
        ],
        out_specs=pl.BlockSpec((1, _R, _NH, _DH),
                               lambda i: (i, 0, 0, 0)),
        out_shape=jax.ShapeDtypeStruct((_L2, _R, _NH, _DH), jnp.float32),
        scratch_shapes=[pltpu.VMEM((_B, _RPB, _PH), jnp.bfloat16)],
        compiler_params=pltpu.CompilerParams(
            dimension_semantics=("arbitrary",)),
    )(g, w2, b2r)


def kernel(tokens, prompt_table, W1, b1, W2, b2):
    idx = jnp.pad(tokens - _V, ((0, 0), (0, _RPB - _P))).reshape(_RPAD)
    a = _mlp1(prompt_table, W1, b1.reshape(1, _PH))
    g = _sc_gather(a, idx)
    out4 = _mlp2(g.reshape(_B, _RPB, _PH), W2, b2.reshape(_L2, 1, _NH * _DH))
    return out4.reshape(_L2, _B, _P, _NH, _DH).transpose(0, 1, 3, 2, 4)
